# baseline (speedup 1.0000x reference)
import functools

import jax
import jax.numpy as jnp
import numpy as np
from jax import lax
from jax.experimental import pallas as pl
from jax.experimental.pallas import tpu as pltpu
from jax.experimental.pallas import tpu_sc as plsc

N_BINS_K = 15
PAD_BINS = 16


def _ece_block_kernel(n_total, n_grid, x_ref, ll_ref, lo_ref, up_ref,
                      out_ref, acc_ref):
    i = pl.program_id(0)
    x = x_ref[...]
    m = jnp.max(x, axis=1, keepdims=True)
    s = jnp.sum(jnp.exp(x - m), axis=1, keepdims=True)
    conf = 1.0 / s
    ll = ll_ref[0, 0, :]
    acc = (ll[:, None] == m).astype(jnp.float32)

    lo = lo_ref[...]
    up = up_ref[...]
    inb = ((conf > lo) & (conf <= up)).astype(jnp.float32)
    cnt = jnp.sum(inb, axis=0, keepdims=True)
    sconf = jnp.sum(inb * conf, axis=0, keepdims=True)
    sacc = jnp.sum(inb * acc, axis=0, keepdims=True)
    upd = jnp.concatenate([cnt, sconf, sacc], axis=0)

    @pl.when(i == 0)
    def _init():
        acc_ref[...] = upd

    @pl.when(i > 0)
    def _accum():
        acc_ref[...] = acc_ref[...] + upd

    @pl.when(i == n_grid - 1)
    def _finish():
        tot = acc_ref[...]
        count = tot[0:1, :]
        tconf = tot[1:2, :]
        tacc = tot[2:3, :]
        denom = jnp.maximum(count, 1.0)
        contrib = jnp.abs(tconf / denom - tacc / denom) * (count / n_total)
        out_ref[...] = jnp.sum(jnp.where(count > 0.0, contrib, 0.0),
                               keepdims=True)


def _gather_label_logits(logits, labels):
    n, c = logits.shape
    info = plsc.get_sparse_core_info()
    nc = info.num_cores
    nw = nc * info.num_subcores
    per = n // nw
    chunks = per // 128
    flat = logits.reshape(-1)
    mesh = plsc.VectorSubcoreMesh(core_axis_name="c", subcore_axis_name="s")

    @functools.partial(
        pl.kernel, mesh=mesh,
        out_type=jax.ShapeDtypeStruct((n,), jnp.float32),
        scratch_types=[
            pltpu.VMEM((per,), jnp.int32),
            pltpu.VMEM((chunks, 128), jnp.int32),
            pltpu.VMEM((per,), jnp.float32),
            pltpu.SemaphoreType.DMA,
        ],
    )
    def k(flat_hbm, labels_hbm, out_hbm, lab_v, idx_v, val_v, sem):
        wid = lax.axis_index("s") * nc + lax.axis_index("c")
        base = wid * per
        pltpu.sync_copy(labels_hbm.at[pl.ds(base, per)], lab_v)
        lane = lax.iota(jnp.int32, 16)
        for j in range(per // 16):
            row0 = base + j * 16
            idx = (row0 + lane) * c + lab_v[pl.ds(j * 16, 16)]
            idx_v[j // 8, pl.ds((j % 8) * 16, 16)] = idx
        copies = [
            pltpu.async_copy(flat_hbm.at[idx_v.at[q]],
                             val_v.at[pl.ds(q * 128, 128)], sem)
            for q in range(chunks)
        ]
        for cp in copies:
            cp.wait()
        pltpu.sync_copy(val_v, out_hbm.at[pl.ds(base, per)])

    return k(flat, labels)


def kernel(logits, labels):
    n, c = logits.shape
    rows = 512
    grid = n // rows
    lablogit = _gather_label_logits(logits, labels)
    lablogit3 = lablogit.reshape(grid, 1, rows)

    bounds = np.linspace(0.0, 1.0, N_BINS_K + 1).astype(np.float32)
    lowers = np.full((1, PAD_BINS), 2.0, np.float32)
    uppers = np.full((1, PAD_BINS), 3.0, np.float32)
    lowers[0, :N_BINS_K] = bounds[:-1]
    uppers[0, :N_BINS_K] = bounds[1:]

    out = pl.pallas_call(
        functools.partial(_ece_block_kernel, float(n), grid),
        grid=(grid,),
        in_specs=[
            pl.BlockSpec((rows, c), lambda i: (i, 0)),
            pl.BlockSpec((1, 1, rows), lambda i: (i, 0, 0)),
            pl.BlockSpec((1, PAD_BINS), lambda i: (0, 0)),
            pl.BlockSpec((1, PAD_BINS), lambda i: (0, 0)),
        ],
        out_specs=pl.BlockSpec((1, 1), lambda i: (0, 0)),
        out_shape=jax.ShapeDtypeStruct((1, 1), jnp.float32),
        scratch_shapes=[pltpu.VMEM((3, PAD_BINS), jnp.float32)],
    )(logits, lablogit3, jnp.asarray(lowers), jnp.asarray(uppers))
    return out.reshape(1)


# speedup vs baseline: 1.0678x; 1.0678x over previous
import functools

import jax
import jax.numpy as jnp
import numpy as np
from jax.experimental import pallas as pl
from jax.experimental.pallas import tpu as pltpu

N_BINS_K = 15
PAD_BINS = 16


def _ece_block_kernel(n_total, n_grid, x_ref, lab_ref, bnd_ref,
                      out_ref, acc_ref):
    i = pl.program_id(0)
    x = x_ref[...]
    r, c = x.shape
    m = jnp.max(x, axis=1, keepdims=True)
    e = jnp.exp(x)
    ones = jnp.ones((c, 1), jnp.float32)
    s = jnp.dot(e, ones, precision=jax.lax.Precision.HIGHEST)
    conf = jnp.minimum(jnp.exp(m) / s, 1.0)
    maskf = (x >= m).astype(jnp.float32)
    col = jax.lax.broadcasted_iota(jnp.int32, (c, 1), 0).astype(jnp.float32)
    predsum = jnp.dot(maskf, col, precision=jax.lax.Precision.HIGHEST)
    labf = lab_ref[0, 0, :].astype(jnp.float32)[:, None]
    acc = (predsum == labf).astype(jnp.float32)

    bnd = bnd_ref[...]
    gt = (conf > bnd).astype(jnp.float32)
    cnt = jnp.sum(gt, axis=0, keepdims=True)
    sconf = jnp.sum(gt * conf, axis=0, keepdims=True)
    sacc = jnp.sum(gt * acc, axis=0, keepdims=True)
    upd = jnp.concatenate([cnt, sconf, sacc], axis=0)

    @pl.when(i == 0)
    def _init():
        acc_ref[...] = upd

    @pl.when(i > 0)
    def _accum():
        acc_ref[...] = acc_ref[...] + upd

    @pl.when(i == n_grid - 1)
    def _finish():
        a = acc_ref[...]
        shifted = jnp.concatenate(
            [a[:, 1:], jnp.zeros((3, 1), jnp.float32)], axis=1)
        b = a - shifted
        count = b[0:1, :]
        tconf = b[1:2, :]
        tacc = b[2:3, :]
        denom = jnp.maximum(count, 1.0)
        contrib = jnp.abs(tconf / denom - tacc / denom) * (count / n_total)
        out_ref[...] = jnp.sum(jnp.where(count > 0.0, contrib, 0.0),
                               keepdims=True)


def kernel(logits, labels):
    n, c = logits.shape
    rows = 512
    grid = n // rows
    labels3 = labels.reshape(grid, 1, rows)

    bounds = np.linspace(0.0, 1.0, N_BINS_K + 1).astype(np.float32)
    bnd = bounds[None, :]

    out = pl.pallas_call(
        functools.partial(_ece_block_kernel, float(n), grid),
        grid=(grid,),
        in_specs=[
            pl.BlockSpec((rows, c), lambda i: (i, 0)),
            pl.BlockSpec((1, 1, rows), lambda i: (i, 0, 0)),
            pl.BlockSpec((1, PAD_BINS), lambda i: (0, 0)),
        ],
        out_specs=pl.BlockSpec((1, 1), lambda i: (0, 0)),
        out_shape=jax.ShapeDtypeStruct((1, 1), jnp.float32),
        scratch_shapes=[pltpu.VMEM((3, PAD_BINS), jnp.float32)],
    )(logits, labels3, jnp.asarray(bnd))
    return out.reshape(1)


# speedup vs baseline: 1.9050x; 1.7841x over previous
import functools

import jax
import jax.numpy as jnp
import numpy as np
from jax.experimental import pallas as pl
from jax.experimental.pallas import tpu as pltpu

N_BINS_K = 15
PAD_BINS = 16


def _ece_block_kernel(n_total, n_grid, x_ref, lab_ref, bnd_ref,
                      out_ref, acc_ref):
    i = pl.program_id(0)
    x = x_ref[...]
    r, c = x.shape
    m = jnp.max(x, axis=1, keepdims=True)
    e = jnp.exp(x)
    s = jnp.sum(e, axis=1, keepdims=True)
    conf = jnp.minimum(jnp.exp(m) / s, 1.0)
    col = jax.lax.broadcasted_iota(jnp.int32, x.shape, 1)
    cand = jnp.where(x == m, col, c)
    pred = jnp.min(cand, axis=1)
    lab = lab_ref[0, 0, :]
    acc = (pred == lab).astype(jnp.float32)[:, None]

    bnd = bnd_ref[...]
    gt = (conf > bnd).astype(jnp.float32)
    cnt = jnp.sum(gt, axis=0, keepdims=True)
    sconf = jnp.sum(gt * conf, axis=0, keepdims=True)
    sacc = jnp.sum(gt * acc, axis=0, keepdims=True)
    upd = jnp.concatenate([cnt, sconf, sacc], axis=0)

    @pl.when(i == 0)
    def _init():
        acc_ref[...] = upd

    @pl.when(i > 0)
    def _accum():
        acc_ref[...] = acc_ref[...] + upd

    @pl.when(i == n_grid - 1)
    def _finish():
        a = acc_ref[...]
        shifted = jnp.concatenate(
            [a[:, 1:], jnp.zeros((3, 1), jnp.float32)], axis=1)
        b = a - shifted
        count = b[0:1, :]
        tconf = b[1:2, :]
        tacc = b[2:3, :]
        denom = jnp.maximum(count, 1.0)
        contrib = jnp.abs(tconf / denom - tacc / denom) * (count / n_total)
        out_ref[...] = jnp.sum(jnp.where(count > 0.0, contrib, 0.0),
                               keepdims=True)


def kernel(logits, labels):
    n, c = logits.shape
    rows = 512
    grid = n // rows
    labels3 = labels.reshape(grid, 1, rows)

    bounds = np.linspace(0.0, 1.0, N_BINS_K + 1).astype(np.float32)
    bnd = bounds[None, :]

    out = pl.pallas_call(
        functools.partial(_ece_block_kernel, float(n), grid),
        grid=(grid,),
        in_specs=[
            pl.BlockSpec((rows, c), lambda i: (i, 0)),
            pl.BlockSpec((1, 1, rows), lambda i: (i, 0, 0)),
            pl.BlockSpec((1, PAD_BINS), lambda i: (0, 0)),
        ],
        out_specs=pl.BlockSpec((1, 1), lambda i: (0, 0)),
        out_shape=jax.ShapeDtypeStruct((1, 1), jnp.float32),
        scratch_shapes=[pltpu.VMEM((3, PAD_BINS), jnp.float32)],
    )(logits, labels3, jnp.asarray(bnd))
    return out.reshape(1)
